# Initial kernel scaffold; baseline (speedup 1.0000x reference)
import jax
import jax.numpy as jnp
from jax.experimental import pallas as pl


def kernel(x, edge_index, pos_encoding, W, b, pe_W, pe_b):
    raise NotImplementedError("write your pallas kernel here")


# speedup vs baseline: 19.2730x; 19.2730x over previous
import functools

import jax
import jax.numpy as jnp
from jax import lax
from jax.experimental import pallas as pl
from jax.experimental.pallas import tpu as pltpu
from jax.experimental.pallas import tpu_sc as plsc

N = 10000
E = 320000
D = 128

NC = 2
NS = 16
NW = NC * NS

CHUNK = 128
NCHUNKS = E // CHUNK

CW = 128
NPAD = 10240
ROWS_PER_TILE_PAD = NPAD // NS
RCHUNKS = ROWS_PER_TILE_PAD // CHUNK

_mesh = plsc.VectorSubcoreMesh(core_axis_name="c", subcore_axis_name="s")


def _worker_chunk_range(wid):
    base = NCHUNKS // NW
    rem = NCHUNKS % NW
    start = wid * base + jnp.minimum(wid, rem)
    cnt = base + jnp.where(wid < rem, 1, 0)
    return start, start + cnt


def _make_hist(cw):
  @functools.partial(
      pl.kernel,
      out_type=jax.ShapeDtypeStruct((NC, NPAD, cw), jnp.float32),
      mesh=_mesh,
      scratch_types=[
          pltpu.VMEM((CHUNK,), jnp.int32),
          pltpu.VMEM((CHUNK, cw), jnp.float32),
          pltpu.VMEM((CHUNK, cw), jnp.float32),
          pltpu.VMEM_SHARED((NPAD, cw), jnp.float32),
      ],
  )
  def _sc_hist(edge_hbm, zc_hbm, ones_hbm, ar_hbm, cnt_hbm, idx_v, ones_v, row_v, cnt_sh):
      c = lax.axis_index("c")
      s = lax.axis_index("s")
      wid = c * NS + s

      pltpu.sync_copy(zc_hbm, row_v)
      pltpu.sync_copy(ones_hbm, ones_v)

      for k in range(RCHUNKS):
          pltpu.sync_copy(ar_hbm.at[pl.ds(s * ROWS_PER_TILE_PAD + k * CHUNK, CHUNK)], idx_v)
          pltpu.sync_copy(row_v, cnt_sh.at[idx_v])

      plsc.subcore_barrier()

      start, end = _worker_chunk_range(wid)

      def body(ch, carry):
          pltpu.sync_copy(edge_hbm.at[1, pl.ds(ch * CHUNK, CHUNK)], idx_v)
          pltpu.sync_copy(ones_v, cnt_sh.at[idx_v], add=True)
          return carry

      lax.fori_loop(start, end, body, None)
      plsc.subcore_barrier()

      for k in range(RCHUNKS):
          r0 = s * ROWS_PER_TILE_PAD + k * CHUNK
          pltpu.sync_copy(ar_hbm.at[pl.ds(r0, CHUNK)], idx_v)
          pltpu.sync_copy(cnt_sh.at[idx_v], row_v)
          pltpu.sync_copy(row_v, cnt_hbm.at[c, pl.ds(r0, CHUNK), :])
  return _sc_hist


_sc_hist = _make_hist(CW)


@functools.partial(
    pl.kernel,
    out_type=jax.ShapeDtypeStruct((NC, NPAD, D), jnp.float32),
    mesh=_mesh,
    scratch_types=[
        pltpu.VMEM((CHUNK,), jnp.int32),
        pltpu.VMEM((CHUNK,), jnp.int32),
        pltpu.VMEM((CHUNK, D), jnp.float32),
        pltpu.VMEM((CHUNK, D), jnp.float32),
        pltpu.VMEM_SHARED((NPAD, D), jnp.float32),
        pltpu.SemaphoreType.DMA,
    ],
)
def _sc_scatter(hp_hbm, edge_hbm, za_hbm, ar_hbm, acc_hbm,
                idx_s, idx_d, rows_v, stage_v, acc_sh, sem):
    c = lax.axis_index("c")
    s = lax.axis_index("s")
    wid = c * NS + s

    pltpu.sync_copy(za_hbm, stage_v)

    for k in range(RCHUNKS):
        pltpu.sync_copy(ar_hbm.at[pl.ds(s * ROWS_PER_TILE_PAD + k * CHUNK, CHUNK)], idx_d)
        pltpu.sync_copy(stage_v, acc_sh.at[idx_d])

    plsc.subcore_barrier()

    start, end = _worker_chunk_range(wid)

    def body(ch, carry):
        base = ch * CHUNK
        pltpu.sync_copy(edge_hbm.at[0, pl.ds(base, CHUNK)], idx_s)
        pltpu.sync_copy(edge_hbm.at[1, pl.ds(base, CHUNK)], idx_d)
        pltpu.async_copy(hp_hbm.at[idx_s], rows_v, sem).wait()
        pltpu.sync_copy(rows_v, acc_sh.at[idx_d], add=True)
        return carry

    lax.fori_loop(start, end, body, None)
    plsc.subcore_barrier()

    for k in range(RCHUNKS):
        r0 = s * ROWS_PER_TILE_PAD + k * CHUNK
        pltpu.sync_copy(ar_hbm.at[pl.ds(r0, CHUNK)], idx_d)
        pltpu.sync_copy(acc_sh.at[idx_d], stage_v)
        pltpu.sync_copy(stage_v, acc_hbm.at[c, pl.ds(r0, CHUNK), :])


def _block_dinv(cnt_ref):
    deg = cnt_ref[0, :, 0:1] + cnt_ref[1, :, 0:1] + 1.0
    return lax.rsqrt(deg)


def _tc_prescale_body(x_ref, w_ref, cnt_ref, hp_ref):
    dinv = _block_dinv(cnt_ref)
    h = jnp.dot(x_ref[...], w_ref[...], preferred_element_type=jnp.float32)
    hp_ref[...] = h * dinv


def _tc_combine_body(acc_ref, hp_ref, cnt_ref, pos_ref, pew_ref, bias_ref, out_ref):
    dinv = _block_dinv(cnt_ref)
    pe = jnp.dot(pos_ref[...], pew_ref[...], preferred_element_type=jnp.float32)
    out_ref[...] = dinv * (acc_ref[0] + acc_ref[1] + hp_ref[...]) + pe + bias_ref[...]


def kernel(x, edge_index, pos_encoding, W, b, pe_W, pe_b):
    zc = jnp.zeros((CHUNK, CW), jnp.float32)
    za = jnp.zeros((CHUNK, D), jnp.float32)
    ones = jnp.ones((CHUNK, CW), jnp.float32)
    ar = jnp.arange(NPAD, dtype=jnp.int32)

    cnt = _sc_hist(edge_index, zc, ones, ar)
    RB = 1000
    grid = N // RB

    hp = pl.pallas_call(
        _tc_prescale_body,
        grid=(grid,),
        in_specs=[
            pl.BlockSpec((RB, D), lambda i: (i, 0)),
            pl.BlockSpec((D, D), lambda i: (0, 0)),
            pl.BlockSpec((NC, RB, CW), lambda i: (0, i, 0)),
        ],
        out_specs=pl.BlockSpec((RB, D), lambda i: (i, 0)),
        out_shape=jax.ShapeDtypeStruct((N, D), jnp.float32),
    )(x, W, cnt)

    acc = _sc_scatter(hp, edge_index, za, ar)

    bias = (b + pe_b).reshape(1, D)
    out = pl.pallas_call(
        _tc_combine_body,
        grid=(grid,),
        in_specs=[
            pl.BlockSpec((NC, RB, D), lambda i: (0, i, 0)),
            pl.BlockSpec((RB, D), lambda i: (i, 0)),
            pl.BlockSpec((NC, RB, CW), lambda i: (0, i, 0)),
            pl.BlockSpec((RB, D), lambda i: (i, 0)),
            pl.BlockSpec((D, D), lambda i: (0, 0)),
            pl.BlockSpec((1, D), lambda i: (0, 0)),
        ],
        out_specs=pl.BlockSpec((RB, D), lambda i: (i, 0)),
        out_shape=jax.ShapeDtypeStruct((N, D), jnp.float32),
    )(acc, hp, cnt, pos_encoding, pe_W, bias)

    return out


# speedup vs baseline: 23.2987x; 1.2089x over previous
import functools

import jax
import jax.numpy as jnp
from jax import lax
from jax.experimental import pallas as pl
from jax.experimental.pallas import tpu as pltpu
from jax.experimental.pallas import tpu_sc as plsc

N = 10000
E = 320000
D = 128

NC = 2
NS = 16
NW = NC * NS

CHUNK = 128
NCHUNKS = E // CHUNK

CW = 128
NPAD = 10240
ROWS_PER_TILE_PAD = NPAD // NS
RCHUNKS = ROWS_PER_TILE_PAD // CHUNK

_mesh = plsc.VectorSubcoreMesh(core_axis_name="c", subcore_axis_name="s")


def _worker_chunk_range(wid):
    base = NCHUNKS // NW
    rem = NCHUNKS % NW
    start = wid * base + jnp.minimum(wid, rem)
    cnt = base + jnp.where(wid < rem, 1, 0)
    return start, start + cnt


def _make_hist(cw):
  @functools.partial(
      pl.kernel,
      out_type=jax.ShapeDtypeStruct((NC, NPAD, cw), jnp.float32),
      mesh=_mesh,
      scratch_types=[
          pltpu.VMEM((CHUNK,), jnp.int32),
          pltpu.VMEM((CHUNK, cw), jnp.float32),
          pltpu.VMEM((CHUNK, cw), jnp.float32),
          pltpu.VMEM_SHARED((NPAD, cw), jnp.float32),
      ],
  )
  def _sc_hist(edge_hbm, zc_hbm, ones_hbm, ar_hbm, cnt_hbm, idx_v, ones_v, row_v, cnt_sh):
      c = lax.axis_index("c")
      s = lax.axis_index("s")
      wid = c * NS + s

      pltpu.sync_copy(zc_hbm, row_v)
      pltpu.sync_copy(ones_hbm, ones_v)

      for k in range(RCHUNKS):
          pltpu.sync_copy(ar_hbm.at[pl.ds(s * ROWS_PER_TILE_PAD + k * CHUNK, CHUNK)], idx_v)
          pltpu.sync_copy(row_v, cnt_sh.at[idx_v])

      plsc.subcore_barrier()

      start, end = _worker_chunk_range(wid)

      def body(ch, carry):
          pltpu.sync_copy(edge_hbm.at[1, pl.ds(ch * CHUNK, CHUNK)], idx_v)
          pltpu.sync_copy(ones_v, cnt_sh.at[idx_v], add=True)
          return carry

      lax.fori_loop(start, end, body, None)
      plsc.subcore_barrier()

      for k in range(RCHUNKS):
          r0 = s * ROWS_PER_TILE_PAD + k * CHUNK
          pltpu.sync_copy(ar_hbm.at[pl.ds(r0, CHUNK)], idx_v)
          pltpu.sync_copy(cnt_sh.at[idx_v], row_v)
          pltpu.sync_copy(row_v, cnt_hbm.at[c, pl.ds(r0, CHUNK), :])
  return _sc_hist


_sc_hist = _make_hist(CW)


@functools.partial(
    pl.kernel,
    out_type=jax.ShapeDtypeStruct((NC, NPAD, D), jnp.float32),
    mesh=_mesh,
    scratch_types=[
        pltpu.VMEM((CHUNK,), jnp.int32),
        pltpu.VMEM((CHUNK,), jnp.int32),
        pltpu.VMEM((CHUNK,), jnp.int32),
        pltpu.VMEM((CHUNK,), jnp.int32),
        pltpu.VMEM((CHUNK, D), jnp.float32),
        pltpu.VMEM((CHUNK, D), jnp.float32),
        pltpu.VMEM_SHARED((NPAD, D), jnp.float32),
        pltpu.SemaphoreType.DMA,
        pltpu.SemaphoreType.DMA,
    ],
)
def _sc_scatter(hp_hbm, edge_hbm, za_hbm, ar_hbm, acc_hbm,
                idx_s, idx_d, idx_s2, idx_d2, rows_v, rows2_v, acc_sh,
                sem, sem2):
    c = lax.axis_index("c")
    s = lax.axis_index("s")
    wid = c * NS + s

    pltpu.sync_copy(za_hbm, rows_v)

    for k in range(RCHUNKS):
        pltpu.sync_copy(ar_hbm.at[pl.ds(s * ROWS_PER_TILE_PAD + k * CHUNK, CHUNK)], idx_d)
        pltpu.sync_copy(rows_v, acc_sh.at[idx_d])

    plsc.subcore_barrier()

    start, end = _worker_chunk_range(wid)
    npairs = (end - start) // 2

    def body(j, carry):
        base0 = (start + 2 * j) * CHUNK
        base1 = base0 + CHUNK
        pltpu.sync_copy(edge_hbm.at[0, pl.ds(base0, CHUNK)], idx_s)
        pltpu.sync_copy(edge_hbm.at[1, pl.ds(base0, CHUNK)], idx_d)
        g0 = pltpu.async_copy(hp_hbm.at[idx_s], rows_v, sem)
        pltpu.sync_copy(edge_hbm.at[0, pl.ds(base1, CHUNK)], idx_s2)
        pltpu.sync_copy(edge_hbm.at[1, pl.ds(base1, CHUNK)], idx_d2)
        g1 = pltpu.async_copy(hp_hbm.at[idx_s2], rows2_v, sem2)
        g0.wait()
        pltpu.sync_copy(rows_v, acc_sh.at[idx_d], add=True)
        g1.wait()
        pltpu.sync_copy(rows2_v, acc_sh.at[idx_d2], add=True)
        return carry

    lax.fori_loop(0, npairs, body, None)

    @pl.when(end - start > 2 * npairs)
    def _():
        base = (end - 1) * CHUNK
        pltpu.sync_copy(edge_hbm.at[0, pl.ds(base, CHUNK)], idx_s)
        pltpu.sync_copy(edge_hbm.at[1, pl.ds(base, CHUNK)], idx_d)
        pltpu.async_copy(hp_hbm.at[idx_s], rows_v, sem).wait()
        pltpu.sync_copy(rows_v, acc_sh.at[idx_d], add=True)

    plsc.subcore_barrier()

    for k in range(RCHUNKS):
        r0 = s * ROWS_PER_TILE_PAD + k * CHUNK
        pltpu.sync_copy(ar_hbm.at[pl.ds(r0, CHUNK)], idx_d)
        pltpu.sync_copy(acc_sh.at[idx_d], rows_v)
        pltpu.sync_copy(rows_v, acc_hbm.at[c, pl.ds(r0, CHUNK), :])


def _block_dinv(cnt_ref):
    deg = cnt_ref[0, :, 0:1] + cnt_ref[1, :, 0:1] + 1.0
    return lax.rsqrt(deg)


def _tc_prescale_body(x_ref, w_ref, cnt_ref, hp_ref):
    dinv = _block_dinv(cnt_ref)
    h = jnp.dot(x_ref[...], w_ref[...], preferred_element_type=jnp.float32)
    hp_ref[...] = h * dinv


def _tc_combine_body(acc_ref, hp_ref, cnt_ref, pos_ref, pew_ref, bias_ref, out_ref):
    dinv = _block_dinv(cnt_ref)
    pe = jnp.dot(pos_ref[...], pew_ref[...], preferred_element_type=jnp.float32)
    out_ref[...] = dinv * (acc_ref[0] + acc_ref[1] + hp_ref[...]) + pe + bias_ref[...]


def kernel(x, edge_index, pos_encoding, W, b, pe_W, pe_b):
    zc = jnp.zeros((CHUNK, CW), jnp.float32)
    za = jnp.zeros((CHUNK, D), jnp.float32)
    ones = jnp.ones((CHUNK, CW), jnp.float32)
    ar = jnp.arange(NPAD, dtype=jnp.int32)

    cnt = _sc_hist(edge_index, zc, ones, ar)
    RB = 1000
    grid = N // RB

    hp = pl.pallas_call(
        _tc_prescale_body,
        grid=(grid,),
        in_specs=[
            pl.BlockSpec((RB, D), lambda i: (i, 0)),
            pl.BlockSpec((D, D), lambda i: (0, 0)),
            pl.BlockSpec((NC, RB, CW), lambda i: (0, i, 0)),
        ],
        out_specs=pl.BlockSpec((RB, D), lambda i: (i, 0)),
        out_shape=jax.ShapeDtypeStruct((N, D), jnp.float32),
    )(x, W, cnt)

    acc = _sc_scatter(hp, edge_index, za, ar)

    bias = (b + pe_b).reshape(1, D)
    out = pl.pallas_call(
        _tc_combine_body,
        grid=(grid,),
        in_specs=[
            pl.BlockSpec((NC, RB, D), lambda i: (0, i, 0)),
            pl.BlockSpec((RB, D), lambda i: (i, 0)),
            pl.BlockSpec((NC, RB, CW), lambda i: (0, i, 0)),
            pl.BlockSpec((RB, D), lambda i: (i, 0)),
            pl.BlockSpec((D, D), lambda i: (0, 0)),
            pl.BlockSpec((1, D), lambda i: (0, 0)),
        ],
        out_specs=pl.BlockSpec((RB, D), lambda i: (i, 0)),
        out_shape=jax.ShapeDtypeStruct((N, D), jnp.float32),
    )(acc, hp, cnt, pos_encoding, pe_W, bias)

    return out
